# baseline (speedup 1.0000x reference)
import jax
import jax.numpy as jnp
from jax import lax
from jax.experimental import pallas as pl
from jax.experimental.pallas import tpu as pltpu
from jax.experimental.pallas import tpu_sc as plsc

N = 10000
E = 320000
D = 128
H = 128
C = 64

NP = 10240
CHUNK = 128
EP = 327680
NCHUNKS = EP // CHUNK
TRIPS = NCHUNKS // 32
NC = 2
NS = 16
NW = NC * NS
RPT = NP // NS

BM = 1280
GRID = NP // BM


def _agg_body(g, src, dst, acc_out, src_v, dst_v, rows_v, acc_sh, sem, dout):
    cid = lax.axis_index("c")
    sid = lax.axis_index("s")
    wid = sid * NC + cid

    nst = dout // 16

    def _zero_rows(i, _):
        r = i // nst
        col = (i % nst) * 16
        rows_v[r, pl.ds(col, 16)] = jnp.zeros((16,), jnp.float32)
        return 0

    lax.fori_loop(0, CHUNK * nst, _zero_rows, 0)

    base_row = pl.multiple_of(sid * RPT, 128)
    for k in range(RPT // CHUNK):
        pltpu.sync_copy(rows_v, acc_sh.at[pl.ds(base_row + k * CHUNK, CHUNK)])

    plsc.subcore_barrier()

    def _step(j, _):
        base = (wid + j * NW) * CHUNK
        pltpu.sync_copy(src.at[pl.ds(base, CHUNK)], src_v)
        cp = pltpu.async_copy(g.at[src_v], rows_v, sem)
        pltpu.sync_copy(dst.at[pl.ds(base, CHUNK)], dst_v)
        cp.wait()
        pltpu.sync_copy(rows_v, acc_sh.at[dst_v], add=True)
        return 0

    lax.fori_loop(0, TRIPS, _step, 0)

    plsc.subcore_barrier()

    for k in range(RPT // CHUNK):
        r0 = base_row + k * CHUNK
        pltpu.sync_copy(acc_sh.at[pl.ds(r0, CHUNK)],
                        acc_out.at[cid, pl.ds(r0, CHUNK)])


def _make_sc_agg(dout):
    mesh = plsc.VectorSubcoreMesh(core_axis_name="c", subcore_axis_name="s")
    scratch = [
        pltpu.VMEM((CHUNK,), jnp.int32),
        pltpu.VMEM((CHUNK,), jnp.int32),
        pltpu.VMEM((CHUNK, dout), jnp.float32),
        pltpu.VMEM_SHARED((NP, dout), jnp.float32),
        pltpu.SemaphoreType.DMA,
    ]

    def body(g, src, dst, acc_out, src_v, dst_v, rows_v, acc_sh, sem):
        _agg_body(g, src, dst, acc_out, src_v, dst_v, rows_v, acc_sh, sem,
                  dout)

    return pl.kernel(body, mesh=mesh,
                     out_type=jax.ShapeDtypeStruct((NC, NP, dout),
                                                   jnp.float32),
                     scratch_types=tuple(scratch))


def _cnt_body(dst, cnt_out, dst_v, ones_v, cnt_sh, sem):
    cid = lax.axis_index("c")
    sid = lax.axis_index("s")
    wid = sid * NC + cid

    def _zero(i, _):
        r = i // 8
        col = (i % 8) * 16
        ones_v[r, pl.ds(col, 16)] = jnp.zeros((16,), jnp.float32)
        return 0

    lax.fori_loop(0, CHUNK * 8, _zero, 0)

    base_row = pl.multiple_of(sid * RPT, 128)
    for k in range(RPT // CHUNK):
        pltpu.sync_copy(ones_v, cnt_sh.at[pl.ds(base_row + k * CHUNK, CHUNK)])

    def _fill(i, _):
        r = i // 8
        col = (i % 8) * 16
        ones_v[r, pl.ds(col, 16)] = jnp.full((16,), 1.0, jnp.float32)
        return 0

    lax.fori_loop(0, CHUNK * 8, _fill, 0)

    plsc.subcore_barrier()

    ntrips = (NCHUNKS - wid + NW - 1) // NW

    def _step(j, _):
        base = (wid + j * NW) * CHUNK
        pltpu.sync_copy(dst.at[pl.ds(base, CHUNK)], dst_v)
        pltpu.sync_copy(ones_v, cnt_sh.at[dst_v], add=True)
        return 0

    lax.fori_loop(0, ntrips, _step, 0)

    plsc.subcore_barrier()

    for k in range(RPT // CHUNK):
        r0 = base_row + k * CHUNK
        pltpu.sync_copy(cnt_sh.at[pl.ds(r0, CHUNK)],
                        cnt_out.at[cid, pl.ds(r0, CHUNK)])


def _make_sc_cnt():
    mesh = plsc.VectorSubcoreMesh(core_axis_name="c", subcore_axis_name="s")
    scratch = [
        pltpu.VMEM((CHUNK,), jnp.int32),
        pltpu.VMEM((CHUNK, H), jnp.float32),
        pltpu.VMEM_SHARED((NP, H), jnp.float32),
        pltpu.SemaphoreType.DMA,
    ]
    return pl.kernel(_cnt_body, mesh=mesh,
                     out_type=jax.ShapeDtypeStruct((NC, NP, H), jnp.float32),
                     scratch_types=tuple(scratch))


_sc_cache = {}


def _sc_agg(dout):
    if dout not in _sc_cache:
        _sc_cache[dout] = _make_sc_agg(dout)
    return _sc_cache[dout]


def _sc_cnt():
    if 'cnt' not in _sc_cache:
        _sc_cache['cnt'] = _make_sc_cnt()
    return _sc_cache['cnt']


def _pre_body(x_ref, wlt_ref, wrt_ref, g_ref, r_ref):
    xb = x_ref[...]
    g_ref[...] = jnp.dot(xb, wlt_ref[...], preferred_element_type=jnp.float32)
    r_ref[...] = jnp.dot(xb, wrt_ref[...], preferred_element_type=jnp.float32)


_pre = pl.pallas_call(
    _pre_body,
    grid=(GRID,),
    in_specs=[
        pl.BlockSpec((BM, D), lambda i: (i, 0)),
        pl.BlockSpec((D, H), lambda i: (0, 0)),
        pl.BlockSpec((D, H), lambda i: (0, 0)),
    ],
    out_specs=[pl.BlockSpec((BM, H), lambda i: (i, 0))] * 2,
    out_shape=[jax.ShapeDtypeStruct((NP, H), jnp.float32)] * 2,
)


def _comb1_body(p0, p1, c0, c1, bl, r, wlt, wrt, g2_ref, r2_ref, invc_ref):
    cnt = c0[...][:, 0:1] + c1[...][:, 0:1]
    invc = 1.0 / jnp.maximum(cnt, 1.0)
    h = jnp.maximum((p0[...] + p1[...]) * invc + bl[...] + r[...], 0.0)
    g2_ref[...] = jnp.dot(h, wlt[...], preferred_element_type=jnp.float32)
    r2_ref[...] = jnp.dot(h, wrt[...], preferred_element_type=jnp.float32)
    invc_ref[...] = jnp.broadcast_to(invc, invc_ref.shape)


_comb1 = pl.pallas_call(
    _comb1_body,
    grid=(GRID,),
    in_specs=[
        pl.BlockSpec((BM, H), lambda i: (i, 0)),
        pl.BlockSpec((BM, H), lambda i: (i, 0)),
        pl.BlockSpec((BM, H), lambda i: (i, 0)),
        pl.BlockSpec((BM, H), lambda i: (i, 0)),
        pl.BlockSpec((1, H), lambda i: (0, 0)),
        pl.BlockSpec((BM, H), lambda i: (i, 0)),
        pl.BlockSpec((H, H), lambda i: (0, 0)),
        pl.BlockSpec((H, H), lambda i: (0, 0)),
    ],
    out_specs=[
        pl.BlockSpec((BM, H), lambda i: (i, 0)),
        pl.BlockSpec((BM, H), lambda i: (i, 0)),
        pl.BlockSpec((BM, H), lambda i: (i, 0)),
    ],
    out_shape=[
        jax.ShapeDtypeStruct((NP, H), jnp.float32),
        jax.ShapeDtypeStruct((NP, H), jnp.float32),
        jax.ShapeDtypeStruct((NP, H), jnp.float32),
    ],
)


def _comb2_body(p0, p1, invc, bl, r, h3_ref):
    h3_ref[...] = jnp.maximum(
        (p0[...] + p1[...]) * invc[...] + bl[...] + r[...], 0.0)


_comb2 = pl.pallas_call(
    _comb2_body,
    grid=(GRID,),
    in_specs=[
        pl.BlockSpec((BM, H), lambda i: (i, 0)),
        pl.BlockSpec((BM, H), lambda i: (i, 0)),
        pl.BlockSpec((BM, H), lambda i: (i, 0)),
        pl.BlockSpec((1, H), lambda i: (0, 0)),
        pl.BlockSpec((BM, H), lambda i: (i, 0)),
    ],
    out_specs=pl.BlockSpec((BM, H), lambda i: (i, 0)),
    out_shape=jax.ShapeDtypeStruct((NP, H), jnp.float32),
)


def _comb3_body(p0, p1, invc, bl, h3, wlt, wrt, out_ref):
    agg = (p0[...] + p1[...]) * invc[...]
    out_ref[...] = jnp.maximum(
        jnp.dot(agg, wlt[...], preferred_element_type=jnp.float32)
        + bl[...]
        + jnp.dot(h3[...], wrt[...], preferred_element_type=jnp.float32),
        0.0)


_comb3 = pl.pallas_call(
    _comb3_body,
    grid=(GRID,),
    in_specs=[
        pl.BlockSpec((BM, H), lambda i: (i, 0)),
        pl.BlockSpec((BM, H), lambda i: (i, 0)),
        pl.BlockSpec((BM, H), lambda i: (i, 0)),
        pl.BlockSpec((1, C), lambda i: (0, 0)),
        pl.BlockSpec((BM, H), lambda i: (i, 0)),
        pl.BlockSpec((H, C), lambda i: (0, 0)),
        pl.BlockSpec((H, C), lambda i: (0, 0)),
    ],
    out_specs=pl.BlockSpec((BM, C), lambda i: (i, 0)),
    out_shape=jax.ShapeDtypeStruct((NP, C), jnp.float32),
)


def kernel(x, edge_index, Wl1, bl1, Wr1, Wl2, bl2, Wr2, Wl3, bl3, Wr3):
    pad = jnp.arange(EP - E, dtype=jnp.int32)
    src = jnp.concatenate([edge_index[0], pad * 0])
    dst = jnp.concatenate([edge_index[1], N + pad % (NP - N)])
    xp = jnp.pad(x, ((0, NP - N), (0, 0)))

    g1, r1 = _pre(xp, Wl1.T, Wr1.T)
    acc1 = _sc_agg(H)(g1, src, dst)
    cnts = _sc_cnt()(dst)
    g2, r2, invc = _comb1(acc1[0], acc1[1], cnts[0], cnts[1],
                          bl1.reshape(1, H), r1, Wl2.T, Wr2.T)
    acc2 = _sc_agg(H)(g2, src, dst)
    h3 = _comb2(acc2[0], acc2[1], invc, bl2.reshape(1, H), r2)
    acc3 = _sc_agg(H)(h3, src, dst)
    out = _comb3(acc3[0], acc3[1], invc, bl3.reshape(1, C), h3,
                 Wl3.T, Wr3.T)
    return out[:N]


# speedup vs baseline: 1.9713x; 1.9713x over previous
import jax
import jax.numpy as jnp
from jax import lax
from jax.experimental import pallas as pl
from jax.experimental.pallas import tpu as pltpu
from jax.experimental.pallas import tpu_sc as plsc

N = 10000
E = 320000
D = 128
H = 128
C = 64

NP = 10240
CHUNK = 128
EP = 327680
NCHUNKS = EP // CHUNK
TRIPS = NCHUNKS // 32
NC = 2
NS = 16
NW = NC * NS
RPT = NP // NS

BM = 1280
GRID = NP // BM


def _agg_body(g, src, dst, acc_out, src_v, dst_v, rows_v, acc_sh, sem, dout):
    cid = lax.axis_index("c")
    sid = lax.axis_index("s")
    wid = sid * NC + cid

    nst = dout // 16

    def _zero_rows(i, _):
        r = i // nst
        col = (i % nst) * 16
        rows_v[r, pl.ds(col, 16)] = jnp.zeros((16,), jnp.float32)
        return 0

    lax.fori_loop(0, CHUNK * nst, _zero_rows, 0)

    base_row = pl.multiple_of(sid * RPT, 128)
    for k in range(RPT // CHUNK):
        pltpu.sync_copy(rows_v, acc_sh.at[pl.ds(base_row + k * CHUNK, CHUNK)])

    plsc.subcore_barrier()

    def _step(j, _):
        base = (wid + j * NW) * CHUNK
        pltpu.sync_copy(src.at[pl.ds(base, CHUNK)], src_v)
        cp = pltpu.async_copy(g.at[src_v], rows_v, sem)
        pltpu.sync_copy(dst.at[pl.ds(base, CHUNK)], dst_v)
        cp.wait()
        pltpu.sync_copy(rows_v, acc_sh.at[dst_v], add=True)
        return 0

    lax.fori_loop(0, TRIPS, _step, 0)

    plsc.subcore_barrier()

    for k in range(RPT // CHUNK):
        r0 = base_row + k * CHUNK
        pltpu.sync_copy(acc_sh.at[pl.ds(r0, CHUNK)],
                        acc_out.at[cid, pl.ds(r0, CHUNK)])


def _make_sc_agg(dout):
    mesh = plsc.VectorSubcoreMesh(core_axis_name="c", subcore_axis_name="s")
    scratch = [
        pltpu.VMEM((CHUNK,), jnp.int32),
        pltpu.VMEM((CHUNK,), jnp.int32),
        pltpu.VMEM((CHUNK, dout), jnp.float32),
        pltpu.VMEM_SHARED((NP, dout), jnp.float32),
        pltpu.SemaphoreType.DMA,
    ]

    def body(g, src, dst, acc_out, src_v, dst_v, rows_v, acc_sh, sem):
        _agg_body(g, src, dst, acc_out, src_v, dst_v, rows_v, acc_sh, sem,
                  dout)

    return pl.kernel(body, mesh=mesh,
                     out_type=jax.ShapeDtypeStruct((NC, NP, dout),
                                                   jnp.float32),
                     scratch_types=tuple(scratch))


def _cnt_body(dst, cnt_out, dst_v, ones_v, cnt_sh, sem):
    cid = lax.axis_index("c")
    sid = lax.axis_index("s")
    wid = sid * NC + cid

    def _zero(i, _):
        r = i // 8
        col = (i % 8) * 16
        ones_v[r, pl.ds(col, 16)] = jnp.zeros((16,), jnp.float32)
        return 0

    lax.fori_loop(0, CHUNK * 8, _zero, 0)

    base_row = pl.multiple_of(sid * RPT, 128)
    for k in range(RPT // CHUNK):
        pltpu.sync_copy(ones_v, cnt_sh.at[pl.ds(base_row + k * CHUNK, CHUNK)])

    def _fill(i, _):
        r = i // 8
        col = (i % 8) * 16
        ones_v[r, pl.ds(col, 16)] = jnp.full((16,), 1.0, jnp.float32)
        return 0

    lax.fori_loop(0, CHUNK * 8, _fill, 0)

    plsc.subcore_barrier()

    ntrips = (NCHUNKS - wid + NW - 1) // NW

    def _step(j, _):
        base = (wid + j * NW) * CHUNK
        pltpu.sync_copy(dst.at[pl.ds(base, CHUNK)], dst_v)
        pltpu.sync_copy(ones_v, cnt_sh.at[dst_v], add=True)
        return 0

    lax.fori_loop(0, ntrips, _step, 0)

    plsc.subcore_barrier()

    for k in range(RPT // CHUNK):
        r0 = base_row + k * CHUNK
        pltpu.sync_copy(cnt_sh.at[pl.ds(r0, CHUNK)],
                        cnt_out.at[cid, pl.ds(r0, CHUNK)])


def _make_sc_cnt():
    mesh = plsc.VectorSubcoreMesh(core_axis_name="c", subcore_axis_name="s")
    scratch = [
        pltpu.VMEM((CHUNK,), jnp.int32),
        pltpu.VMEM((CHUNK, H), jnp.float32),
        pltpu.VMEM_SHARED((NP, H), jnp.float32),
        pltpu.SemaphoreType.DMA,
    ]
    return pl.kernel(_cnt_body, mesh=mesh,
                     out_type=jax.ShapeDtypeStruct((NC, NP, H), jnp.float32),
                     scratch_types=tuple(scratch))


_sc_cache = {}


def _sc_agg(dout):
    if dout not in _sc_cache:
        _sc_cache[dout] = _make_sc_agg(dout)
    return _sc_cache[dout]


def _sc_cnt():
    if 'cnt' not in _sc_cache:
        _sc_cache['cnt'] = _make_sc_cnt()
    return _sc_cache['cnt']


def _pre_body(x_ref, wlt_ref, wrt_ref, g_ref, r_ref):
    xb = x_ref[...]
    g_ref[...] = jnp.dot(xb, wlt_ref[...], preferred_element_type=jnp.float32)
    r_ref[...] = jnp.dot(xb, wrt_ref[...], preferred_element_type=jnp.float32)


_pre = pl.pallas_call(
    _pre_body,
    grid=(GRID,),
    in_specs=[
        pl.BlockSpec((BM, D), lambda i: (i, 0)),
        pl.BlockSpec((D, H), lambda i: (0, 0)),
        pl.BlockSpec((D, H), lambda i: (0, 0)),
    ],
    out_specs=[pl.BlockSpec((BM, H), lambda i: (i, 0))] * 2,
    out_shape=[jax.ShapeDtypeStruct((NP, H), jnp.float32)] * 2,
)


def _comb1_body(p0, p1, c0, c1, bl, r, wlt, wrt, g2_ref, r2_ref, invc_ref):
    cnt = c0[...][:, 0:1] + c1[...][:, 0:1]
    invc = 1.0 / jnp.maximum(cnt, 1.0)
    h = jnp.maximum((p0[...] + p1[...]) * invc + bl[...] + r[...], 0.0)
    g2_ref[...] = jnp.dot(h, wlt[...], preferred_element_type=jnp.float32)
    r2_ref[...] = jnp.dot(h, wrt[...], preferred_element_type=jnp.float32)
    invc_ref[...] = jnp.broadcast_to(invc, invc_ref.shape)


_comb1 = pl.pallas_call(
    _comb1_body,
    grid=(GRID,),
    in_specs=[
        pl.BlockSpec((BM, H), lambda i: (i, 0)),
        pl.BlockSpec((BM, H), lambda i: (i, 0)),
        pl.BlockSpec((BM, H), lambda i: (i, 0)),
        pl.BlockSpec((BM, H), lambda i: (i, 0)),
        pl.BlockSpec((1, H), lambda i: (0, 0)),
        pl.BlockSpec((BM, H), lambda i: (i, 0)),
        pl.BlockSpec((H, H), lambda i: (0, 0)),
        pl.BlockSpec((H, H), lambda i: (0, 0)),
    ],
    out_specs=[
        pl.BlockSpec((BM, H), lambda i: (i, 0)),
        pl.BlockSpec((BM, H), lambda i: (i, 0)),
        pl.BlockSpec((BM, H), lambda i: (i, 0)),
    ],
    out_shape=[
        jax.ShapeDtypeStruct((NP, H), jnp.float32),
        jax.ShapeDtypeStruct((NP, H), jnp.float32),
        jax.ShapeDtypeStruct((NP, H), jnp.float32),
    ],
)


def _comb2_body(p0, p1, invc, bl, r, h3_ref):
    h3_ref[...] = jnp.maximum(
        (p0[...] + p1[...]) * invc[...] + bl[...] + r[...], 0.0)


_comb2 = pl.pallas_call(
    _comb2_body,
    grid=(GRID,),
    in_specs=[
        pl.BlockSpec((BM, H), lambda i: (i, 0)),
        pl.BlockSpec((BM, H), lambda i: (i, 0)),
        pl.BlockSpec((BM, H), lambda i: (i, 0)),
        pl.BlockSpec((1, H), lambda i: (0, 0)),
        pl.BlockSpec((BM, H), lambda i: (i, 0)),
    ],
    out_specs=pl.BlockSpec((BM, H), lambda i: (i, 0)),
    out_shape=jax.ShapeDtypeStruct((NP, H), jnp.float32),
)


def _comb3_body(p0, p1, invc, bl, h3, wlt, wrt, out_ref):
    agg = (p0[...] + p1[...]) * invc[...]
    out_ref[...] = jnp.maximum(
        jnp.dot(agg, wlt[...], preferred_element_type=jnp.float32)
        + bl[...]
        + jnp.dot(h3[...], wrt[...], preferred_element_type=jnp.float32),
        0.0)


_comb3 = pl.pallas_call(
    _comb3_body,
    grid=(GRID,),
    in_specs=[
        pl.BlockSpec((BM, H), lambda i: (i, 0)),
        pl.BlockSpec((BM, H), lambda i: (i, 0)),
        pl.BlockSpec((BM, H), lambda i: (i, 0)),
        pl.BlockSpec((1, C), lambda i: (0, 0)),
        pl.BlockSpec((BM, H), lambda i: (i, 0)),
        pl.BlockSpec((H, C), lambda i: (0, 0)),
        pl.BlockSpec((H, C), lambda i: (0, 0)),
    ],
    out_specs=pl.BlockSpec((BM, C), lambda i: (i, 0)),
    out_shape=jax.ShapeDtypeStruct((NP, C), jnp.float32),
)


def kernel(x, edge_index, Wl1, bl1, Wr1, Wl2, bl2, Wr2, Wl3, bl3, Wr3):
    pad = jnp.arange(EP - E, dtype=jnp.int32)
    src = jnp.concatenate([edge_index[0], pad % N])
    dst = jnp.concatenate([edge_index[1], N + pad % (NP - N)])
    xp = jnp.pad(x, ((0, NP - N), (0, 0)))

    g1, r1 = _pre(xp, Wl1.T, Wr1.T)
    acc1 = _sc_agg(H)(g1, src, dst)
    cnts = _sc_cnt()(dst)
    g2, r2, invc = _comb1(acc1[0], acc1[1], cnts[0], cnts[1],
                          bl1.reshape(1, H), r1, Wl2.T, Wr2.T)
    acc2 = _sc_agg(H)(g2, src, dst)
    h3 = _comb2(acc2[0], acc2[1], invc, bl2.reshape(1, H), r2)
    acc3 = _sc_agg(H)(h3, src, dst)
    out = _comb3(acc3[0], acc3[1], invc, bl3.reshape(1, C), h3,
                 Wl3.T, Wr3.T)
    return out[:N]


# speedup vs baseline: 2.3756x; 1.2051x over previous
import jax
import jax.numpy as jnp
from jax import lax
from jax.experimental import pallas as pl
from jax.experimental.pallas import tpu as pltpu
from jax.experimental.pallas import tpu_sc as plsc

N = 10000
E = 320000
D = 128
H = 128
C = 64

NP = 10240
CHUNK = 128
EP = 327680
NCHUNKS = EP // CHUNK
TRIPS = NCHUNKS // 32
NC = 2
NS = 16
NW = NC * NS
RPT = NP // NS

BM = 1280
GRID = NP // BM


def _agg_body(g, src, dst, acc_out, src_v0, dst_v0, rows_v0,
              src_v1, dst_v1, rows_v1, acc_sh, sem, dout):
    cid = lax.axis_index("c")
    sid = lax.axis_index("s")
    wid = sid * NC + cid

    nst = dout // 16

    def _zero_rows(i, _):
        r = i // nst
        col = (i % nst) * 16
        rows_v0[r, pl.ds(col, 16)] = jnp.zeros((16,), jnp.float32)
        return 0

    lax.fori_loop(0, CHUNK * nst, _zero_rows, 0)

    base_row = pl.multiple_of(sid * RPT, 128)
    for k in range(RPT // CHUNK):
        pltpu.sync_copy(rows_v0,
                        acc_sh.at[pl.ds(base_row + k * CHUNK, CHUNK)])

    plsc.subcore_barrier()

    b0 = (src_v0, dst_v0, rows_v0)
    b1 = (src_v1, dst_v1, rows_v1)

    def _issue(buf, j):
        sv, dv, rv = buf
        base = (wid + j * NW) * CHUNK
        pltpu.sync_copy(src.at[pl.ds(base, CHUNK)], sv)
        pltpu.async_copy(g.at[sv], rv, sem)
        pltpu.sync_copy(dst.at[pl.ds(base, CHUNK)], dv)

    def _wait(buf):
        sv, dv, rv = buf
        pltpu.make_async_copy(g.at[sv], rv, sem).wait()

    def _scat(buf):
        sv, dv, rv = buf
        pltpu.sync_copy(rv, acc_sh.at[dv], add=True)

    _issue(b0, 0)

    def _pair(jj, _):
        _wait(b0)
        _issue(b1, 2 * jj + 1)
        _scat(b0)
        _wait(b1)
        _issue(b0, 2 * jj + 2)
        _scat(b1)
        return 0

    lax.fori_loop(0, TRIPS // 2 - 1, _pair, 0)

    _wait(b0)
    _issue(b1, TRIPS - 1)
    _scat(b0)
    _wait(b1)
    _scat(b1)

    plsc.subcore_barrier()

    for k in range(RPT // CHUNK):
        r0 = base_row + k * CHUNK
        pltpu.sync_copy(acc_sh.at[pl.ds(r0, CHUNK)],
                        acc_out.at[cid, pl.ds(r0, CHUNK)])


def _make_sc_agg(dout):
    mesh = plsc.VectorSubcoreMesh(core_axis_name="c", subcore_axis_name="s")
    scratch = [
        pltpu.VMEM((CHUNK,), jnp.int32),
        pltpu.VMEM((CHUNK,), jnp.int32),
        pltpu.VMEM((CHUNK, dout), jnp.float32),
        pltpu.VMEM((CHUNK,), jnp.int32),
        pltpu.VMEM((CHUNK,), jnp.int32),
        pltpu.VMEM((CHUNK, dout), jnp.float32),
        pltpu.VMEM_SHARED((NP, dout), jnp.float32),
        pltpu.SemaphoreType.DMA,
    ]

    def body(g, src, dst, acc_out, src_v0, dst_v0, rows_v0,
             src_v1, dst_v1, rows_v1, acc_sh, sem):
        _agg_body(g, src, dst, acc_out, src_v0, dst_v0, rows_v0,
                  src_v1, dst_v1, rows_v1, acc_sh, sem, dout)

    return pl.kernel(body, mesh=mesh,
                     out_type=jax.ShapeDtypeStruct((NC, NP, dout),
                                                   jnp.float32),
                     scratch_types=tuple(scratch))


def _cnt_body(dst, cnt_out, dst_v, ones_v, cnt_sh, sem):
    cid = lax.axis_index("c")
    sid = lax.axis_index("s")
    wid = sid * NC + cid

    def _zero(i, _):
        r = i // 8
        col = (i % 8) * 16
        ones_v[r, pl.ds(col, 16)] = jnp.zeros((16,), jnp.float32)
        return 0

    lax.fori_loop(0, CHUNK * 8, _zero, 0)

    base_row = pl.multiple_of(sid * RPT, 128)
    for k in range(RPT // CHUNK):
        pltpu.sync_copy(ones_v, cnt_sh.at[pl.ds(base_row + k * CHUNK, CHUNK)])

    def _fill(i, _):
        r = i // 8
        col = (i % 8) * 16
        ones_v[r, pl.ds(col, 16)] = jnp.full((16,), 1.0, jnp.float32)
        return 0

    lax.fori_loop(0, CHUNK * 8, _fill, 0)

    plsc.subcore_barrier()

    def _step(j, _):
        base = (wid + j * NW) * CHUNK
        pltpu.sync_copy(dst.at[pl.ds(base, CHUNK)], dst_v)
        pltpu.sync_copy(ones_v, cnt_sh.at[dst_v], add=True)
        return 0

    lax.fori_loop(0, TRIPS, _step, 0)

    plsc.subcore_barrier()

    for k in range(RPT // CHUNK):
        r0 = base_row + k * CHUNK
        pltpu.sync_copy(cnt_sh.at[pl.ds(r0, CHUNK)],
                        cnt_out.at[cid, pl.ds(r0, CHUNK)])


def _make_sc_cnt():
    mesh = plsc.VectorSubcoreMesh(core_axis_name="c", subcore_axis_name="s")
    scratch = [
        pltpu.VMEM((CHUNK,), jnp.int32),
        pltpu.VMEM((CHUNK, H), jnp.float32),
        pltpu.VMEM_SHARED((NP, H), jnp.float32),
        pltpu.SemaphoreType.DMA,
    ]
    return pl.kernel(_cnt_body, mesh=mesh,
                     out_type=jax.ShapeDtypeStruct((NC, NP, H), jnp.float32),
                     scratch_types=tuple(scratch))


_sc_cache = {}


def _sc_agg(dout):
    if dout not in _sc_cache:
        _sc_cache[dout] = _make_sc_agg(dout)
    return _sc_cache[dout]


def _sc_cnt():
    if 'cnt' not in _sc_cache:
        _sc_cache['cnt'] = _make_sc_cnt()
    return _sc_cache['cnt']


def _pre_body(x_ref, wlt_ref, wrt_ref, g_ref, r_ref):
    xb = x_ref[...]
    g_ref[...] = jnp.dot(xb, wlt_ref[...], preferred_element_type=jnp.float32)
    r_ref[...] = jnp.dot(xb, wrt_ref[...], preferred_element_type=jnp.float32)


_pre = pl.pallas_call(
    _pre_body,
    grid=(GRID,),
    in_specs=[
        pl.BlockSpec((BM, D), lambda i: (i, 0)),
        pl.BlockSpec((D, H), lambda i: (0, 0)),
        pl.BlockSpec((D, H), lambda i: (0, 0)),
    ],
    out_specs=[pl.BlockSpec((BM, H), lambda i: (i, 0))] * 2,
    out_shape=[jax.ShapeDtypeStruct((NP, H), jnp.float32)] * 2,
)


def _comb1_body(p0, p1, c0, c1, bl, r, wlt, wrt, g2_ref, r2_ref, invc_ref):
    cnt = c0[...][:, 0:1] + c1[...][:, 0:1]
    invc = 1.0 / jnp.maximum(cnt, 1.0)
    h = jnp.maximum((p0[...] + p1[...]) * invc + bl[...] + r[...], 0.0)
    g2_ref[...] = jnp.dot(h, wlt[...], preferred_element_type=jnp.float32)
    r2_ref[...] = jnp.dot(h, wrt[...], preferred_element_type=jnp.float32)
    invc_ref[...] = jnp.broadcast_to(invc, invc_ref.shape)


_comb1 = pl.pallas_call(
    _comb1_body,
    grid=(GRID,),
    in_specs=[
        pl.BlockSpec((BM, H), lambda i: (i, 0)),
        pl.BlockSpec((BM, H), lambda i: (i, 0)),
        pl.BlockSpec((BM, H), lambda i: (i, 0)),
        pl.BlockSpec((BM, H), lambda i: (i, 0)),
        pl.BlockSpec((1, H), lambda i: (0, 0)),
        pl.BlockSpec((BM, H), lambda i: (i, 0)),
        pl.BlockSpec((H, H), lambda i: (0, 0)),
        pl.BlockSpec((H, H), lambda i: (0, 0)),
    ],
    out_specs=[
        pl.BlockSpec((BM, H), lambda i: (i, 0)),
        pl.BlockSpec((BM, H), lambda i: (i, 0)),
        pl.BlockSpec((BM, H), lambda i: (i, 0)),
    ],
    out_shape=[
        jax.ShapeDtypeStruct((NP, H), jnp.float32),
        jax.ShapeDtypeStruct((NP, H), jnp.float32),
        jax.ShapeDtypeStruct((NP, H), jnp.float32),
    ],
)


def _comb2_body(p0, p1, invc, bl, r, h3_ref):
    h3_ref[...] = jnp.maximum(
        (p0[...] + p1[...]) * invc[...] + bl[...] + r[...], 0.0)


_comb2 = pl.pallas_call(
    _comb2_body,
    grid=(GRID,),
    in_specs=[
        pl.BlockSpec((BM, H), lambda i: (i, 0)),
        pl.BlockSpec((BM, H), lambda i: (i, 0)),
        pl.BlockSpec((BM, H), lambda i: (i, 0)),
        pl.BlockSpec((1, H), lambda i: (0, 0)),
        pl.BlockSpec((BM, H), lambda i: (i, 0)),
    ],
    out_specs=pl.BlockSpec((BM, H), lambda i: (i, 0)),
    out_shape=jax.ShapeDtypeStruct((NP, H), jnp.float32),
)


def _comb3_body(p0, p1, invc, bl, h3, wlt, wrt, out_ref):
    agg = (p0[...] + p1[...]) * invc[...]
    out_ref[...] = jnp.maximum(
        jnp.dot(agg, wlt[...], preferred_element_type=jnp.float32)
        + bl[...]
        + jnp.dot(h3[...], wrt[...], preferred_element_type=jnp.float32),
        0.0)


_comb3 = pl.pallas_call(
    _comb3_body,
    grid=(GRID,),
    in_specs=[
        pl.BlockSpec((BM, H), lambda i: (i, 0)),
        pl.BlockSpec((BM, H), lambda i: (i, 0)),
        pl.BlockSpec((BM, H), lambda i: (i, 0)),
        pl.BlockSpec((1, C), lambda i: (0, 0)),
        pl.BlockSpec((BM, H), lambda i: (i, 0)),
        pl.BlockSpec((H, C), lambda i: (0, 0)),
        pl.BlockSpec((H, C), lambda i: (0, 0)),
    ],
    out_specs=pl.BlockSpec((BM, C), lambda i: (i, 0)),
    out_shape=jax.ShapeDtypeStruct((NP, C), jnp.float32),
)


def kernel(x, edge_index, Wl1, bl1, Wr1, Wl2, bl2, Wr2, Wl3, bl3, Wr3):
    pad = jnp.arange(EP - E, dtype=jnp.int32)
    src = jnp.concatenate([edge_index[0], pad % N])
    dst = jnp.concatenate([edge_index[1], N + pad % (NP - N)])
    xp = jnp.pad(x, ((0, NP - N), (0, 0)))

    g1, r1 = _pre(xp, Wl1.T, Wr1.T)
    acc1 = _sc_agg(H)(g1, src, dst)
    cnts = _sc_cnt()(dst)
    g2, r2, invc = _comb1(acc1[0], acc1[1], cnts[0], cnts[1],
                          bl1.reshape(1, H), r1, Wl2.T, Wr2.T)
    acc2 = _sc_agg(H)(g2, src, dst)
    h3 = _comb2(acc2[0], acc2[1], invc, bl2.reshape(1, H), r2)
    acc3 = _sc_agg(H)(h3, src, dst)
    out = _comb3(acc3[0], acc3[1], invc, bl3.reshape(1, C), h3,
                 Wl3.T, Wr3.T)
    return out[:N]


# speedup vs baseline: 2.4663x; 1.0382x over previous
import jax
import jax.numpy as jnp
from jax import lax
from jax.experimental import pallas as pl
from jax.experimental.pallas import tpu as pltpu
from jax.experimental.pallas import tpu_sc as plsc

N = 10000
E = 320000
D = 128
H = 128
C = 64

NP = 10240
CHUNK = 128
EP = 327680
NCHUNKS = EP // CHUNK
TRIPS = NCHUNKS // 32
NC = 2
NS = 16
NW = NC * NS
RPT = NP // NS

BM = 1280
GRID = NP // BM


def _agg_body(g, src, dst, acc_out, didx, sv0, sv1, rows_v0, rows_v1,
              acc_sh, sem, dout):
    cid = lax.axis_index("c")
    sid = lax.axis_index("s")
    wid = sid * NC + cid

    c0 = pl.multiple_of(wid * TRIPS, 8)
    pltpu.sync_copy(dst.at[pl.ds(c0, TRIPS)], didx)

    nst = dout // 16

    def _zero_rows(i, _):
        r = i // nst
        col = (i % nst) * 16
        rows_v0[r, pl.ds(col, 16)] = jnp.zeros((16,), jnp.float32)
        return 0

    lax.fori_loop(0, CHUNK * nst, _zero_rows, 0)

    base_row = pl.multiple_of(sid * RPT, 128)
    for k in range(RPT // CHUNK):
        pltpu.sync_copy(rows_v0,
                        acc_sh.at[pl.ds(base_row + k * CHUNK, CHUNK)])

    plsc.subcore_barrier()

    def _issue(sv, rv, j):
        pltpu.sync_copy(src.at[c0 + j], sv)
        pltpu.async_copy(g.at[sv], rv, sem)

    def _wait(sv, rv):
        pltpu.make_async_copy(g.at[sv], rv, sem).wait()

    def _scat(rv, j):
        pltpu.sync_copy(rv, acc_sh.at[didx.at[j]], add=True)

    _issue(sv0, rows_v0, 0)

    def _pair(jj, _):
        j = 2 * jj
        _wait(sv0, rows_v0)
        _issue(sv1, rows_v1, j + 1)
        _scat(rows_v0, j)
        _wait(sv1, rows_v1)
        _issue(sv0, rows_v0, j + 2)
        _scat(rows_v1, j + 1)
        return 0

    lax.fori_loop(0, TRIPS // 2 - 1, _pair, 0)

    _wait(sv0, rows_v0)
    _issue(sv1, rows_v1, TRIPS - 1)
    _scat(rows_v0, TRIPS - 2)
    _wait(sv1, rows_v1)
    _scat(rows_v1, TRIPS - 1)

    plsc.subcore_barrier()

    for k in range(RPT // CHUNK):
        r0 = base_row + k * CHUNK
        pltpu.sync_copy(acc_sh.at[pl.ds(r0, CHUNK)],
                        acc_out.at[cid, pl.ds(r0, CHUNK)])


def _make_sc_agg(dout):
    mesh = plsc.VectorSubcoreMesh(core_axis_name="c", subcore_axis_name="s")
    scratch = [
        pltpu.VMEM((TRIPS, CHUNK), jnp.int32),
        pltpu.VMEM((CHUNK,), jnp.int32),
        pltpu.VMEM((CHUNK,), jnp.int32),
        pltpu.VMEM((CHUNK, dout), jnp.float32),
        pltpu.VMEM((CHUNK, dout), jnp.float32),
        pltpu.VMEM_SHARED((NP, dout), jnp.float32),
        pltpu.SemaphoreType.DMA,
    ]

    def body(g, src, dst, acc_out, didx, sv0, sv1, rows_v0, rows_v1,
             acc_sh, sem):
        _agg_body(g, src, dst, acc_out, didx, sv0, sv1, rows_v0, rows_v1,
                  acc_sh, sem, dout)

    return pl.kernel(body, mesh=mesh,
                     out_type=jax.ShapeDtypeStruct((NC, NP, dout),
                                                   jnp.float32),
                     scratch_types=tuple(scratch))


def _cnt_body(dst, cnt_out, didx, ones_v, cnt_sh, sem):
    cid = lax.axis_index("c")
    sid = lax.axis_index("s")
    wid = sid * NC + cid

    c0 = pl.multiple_of(wid * TRIPS, 8)
    pltpu.sync_copy(dst.at[pl.ds(c0, TRIPS)], didx)

    def _zero(i, _):
        r = i // 8
        col = (i % 8) * 16
        ones_v[r, pl.ds(col, 16)] = jnp.zeros((16,), jnp.float32)
        return 0

    lax.fori_loop(0, CHUNK * 8, _zero, 0)

    base_row = pl.multiple_of(sid * RPT, 128)
    for k in range(RPT // CHUNK):
        pltpu.sync_copy(ones_v, cnt_sh.at[pl.ds(base_row + k * CHUNK, CHUNK)])

    def _fill(i, _):
        r = i // 8
        col = (i % 8) * 16
        ones_v[r, pl.ds(col, 16)] = jnp.full((16,), 1.0, jnp.float32)
        return 0

    lax.fori_loop(0, CHUNK * 8, _fill, 0)

    plsc.subcore_barrier()

    def _step(j, _):
        pltpu.sync_copy(ones_v, cnt_sh.at[didx.at[j]], add=True)
        return 0

    lax.fori_loop(0, TRIPS, _step, 0)

    plsc.subcore_barrier()

    for k in range(RPT // CHUNK):
        r0 = base_row + k * CHUNK
        pltpu.sync_copy(cnt_sh.at[pl.ds(r0, CHUNK)],
                        cnt_out.at[cid, pl.ds(r0, CHUNK)])


def _make_sc_cnt():
    mesh = plsc.VectorSubcoreMesh(core_axis_name="c", subcore_axis_name="s")
    scratch = [
        pltpu.VMEM((TRIPS, CHUNK), jnp.int32),
        pltpu.VMEM((CHUNK, H), jnp.float32),
        pltpu.VMEM_SHARED((NP, H), jnp.float32),
        pltpu.SemaphoreType.DMA,
    ]
    return pl.kernel(_cnt_body, mesh=mesh,
                     out_type=jax.ShapeDtypeStruct((NC, NP, H), jnp.float32),
                     scratch_types=tuple(scratch))


_sc_cache = {}


def _sc_agg(dout):
    if dout not in _sc_cache:
        _sc_cache[dout] = _make_sc_agg(dout)
    return _sc_cache[dout]


def _sc_cnt():
    if 'cnt' not in _sc_cache:
        _sc_cache['cnt'] = _make_sc_cnt()
    return _sc_cache['cnt']


def _pre_body(x_ref, wlt_ref, wrt_ref, g_ref, r_ref):
    xb = x_ref[...]
    g_ref[...] = jnp.dot(xb, wlt_ref[...], preferred_element_type=jnp.float32)
    r_ref[...] = jnp.dot(xb, wrt_ref[...], preferred_element_type=jnp.float32)


_pre = pl.pallas_call(
    _pre_body,
    grid=(GRID,),
    in_specs=[
        pl.BlockSpec((BM, D), lambda i: (i, 0)),
        pl.BlockSpec((D, H), lambda i: (0, 0)),
        pl.BlockSpec((D, H), lambda i: (0, 0)),
    ],
    out_specs=[pl.BlockSpec((BM, H), lambda i: (i, 0))] * 2,
    out_shape=[jax.ShapeDtypeStruct((NP, H), jnp.float32)] * 2,
)


def _comb1_body(p0, p1, c0, c1, bl, r, wlt, wrt, g2_ref, r2_ref, invc_ref):
    cnt = c0[...][:, 0:1] + c1[...][:, 0:1]
    invc = 1.0 / jnp.maximum(cnt, 1.0)
    h = jnp.maximum((p0[...] + p1[...]) * invc + bl[...] + r[...], 0.0)
    g2_ref[...] = jnp.dot(h, wlt[...], preferred_element_type=jnp.float32)
    r2_ref[...] = jnp.dot(h, wrt[...], preferred_element_type=jnp.float32)
    invc_ref[...] = jnp.broadcast_to(invc, invc_ref.shape)


_comb1 = pl.pallas_call(
    _comb1_body,
    grid=(GRID,),
    in_specs=[
        pl.BlockSpec((BM, H), lambda i: (i, 0)),
        pl.BlockSpec((BM, H), lambda i: (i, 0)),
        pl.BlockSpec((BM, H), lambda i: (i, 0)),
        pl.BlockSpec((BM, H), lambda i: (i, 0)),
        pl.BlockSpec((1, H), lambda i: (0, 0)),
        pl.BlockSpec((BM, H), lambda i: (i, 0)),
        pl.BlockSpec((H, H), lambda i: (0, 0)),
        pl.BlockSpec((H, H), lambda i: (0, 0)),
    ],
    out_specs=[
        pl.BlockSpec((BM, H), lambda i: (i, 0)),
        pl.BlockSpec((BM, H), lambda i: (i, 0)),
        pl.BlockSpec((BM, H), lambda i: (i, 0)),
    ],
    out_shape=[
        jax.ShapeDtypeStruct((NP, H), jnp.float32),
        jax.ShapeDtypeStruct((NP, H), jnp.float32),
        jax.ShapeDtypeStruct((NP, H), jnp.float32),
    ],
)


def _comb2_body(p0, p1, invc, bl, r, h3_ref):
    h3_ref[...] = jnp.maximum(
        (p0[...] + p1[...]) * invc[...] + bl[...] + r[...], 0.0)


_comb2 = pl.pallas_call(
    _comb2_body,
    grid=(GRID,),
    in_specs=[
        pl.BlockSpec((BM, H), lambda i: (i, 0)),
        pl.BlockSpec((BM, H), lambda i: (i, 0)),
        pl.BlockSpec((BM, H), lambda i: (i, 0)),
        pl.BlockSpec((1, H), lambda i: (0, 0)),
        pl.BlockSpec((BM, H), lambda i: (i, 0)),
    ],
    out_specs=pl.BlockSpec((BM, H), lambda i: (i, 0)),
    out_shape=jax.ShapeDtypeStruct((NP, H), jnp.float32),
)


def _comb3_body(p0, p1, invc, bl, h3, wlt, wrt, out_ref):
    agg = (p0[...] + p1[...]) * invc[...]
    out_ref[...] = jnp.maximum(
        jnp.dot(agg, wlt[...], preferred_element_type=jnp.float32)
        + bl[...]
        + jnp.dot(h3[...], wrt[...], preferred_element_type=jnp.float32),
        0.0)


_comb3 = pl.pallas_call(
    _comb3_body,
    grid=(GRID,),
    in_specs=[
        pl.BlockSpec((BM, H), lambda i: (i, 0)),
        pl.BlockSpec((BM, H), lambda i: (i, 0)),
        pl.BlockSpec((BM, H), lambda i: (i, 0)),
        pl.BlockSpec((1, C), lambda i: (0, 0)),
        pl.BlockSpec((BM, H), lambda i: (i, 0)),
        pl.BlockSpec((H, C), lambda i: (0, 0)),
        pl.BlockSpec((H, C), lambda i: (0, 0)),
    ],
    out_specs=pl.BlockSpec((BM, C), lambda i: (i, 0)),
    out_shape=jax.ShapeDtypeStruct((NP, C), jnp.float32),
)


def kernel(x, edge_index, Wl1, bl1, Wr1, Wl2, bl2, Wr2, Wl3, bl3, Wr3):
    pad = jnp.arange(EP - E, dtype=jnp.int32)
    src = jnp.concatenate([edge_index[0], pad % N]).reshape(NCHUNKS, CHUNK)
    dst = jnp.concatenate(
        [edge_index[1], N + pad % (NP - N)]).reshape(NCHUNKS, CHUNK)
    xp = jnp.pad(x, ((0, NP - N), (0, 0)))

    g1, r1 = _pre(xp, Wl1.T, Wr1.T)
    acc1 = _sc_agg(H)(g1, src, dst)
    cnts = _sc_cnt()(dst)
    g2, r2, invc = _comb1(acc1[0], acc1[1], cnts[0], cnts[1],
                          bl1.reshape(1, H), r1, Wl2.T, Wr2.T)
    acc2 = _sc_agg(H)(g2, src, dst)
    h3 = _comb2(acc2[0], acc2[1], invc, bl2.reshape(1, H), r2)
    acc3 = _sc_agg(H)(h3, src, dst)
    out = _comb3(acc3[0], acc3[1], invc, bl3.reshape(1, C), h3,
                 Wl3.T, Wr3.T)
    return out[:N]


# speedup vs baseline: 2.4712x; 1.0020x over previous
import jax
import jax.numpy as jnp
from jax import lax
from jax.experimental import pallas as pl
from jax.experimental.pallas import tpu as pltpu
from jax.experimental.pallas import tpu_sc as plsc

N = 10000
E = 320000
D = 128
H = 128
C = 64

NP = 10240
CHUNK = 128
EP = 327680
NCHUNKS = EP // CHUNK
TRIPS = NCHUNKS // 32
NC = 2
NS = 16
NW = NC * NS
RPT = NP // NS

BM = 1280
GRID = NP // BM


def _agg_body(g, src, dst, acc_out, didx, sv0, sv1, rows_v0, rows_v1,
              acc_sh, sem, sem2, dout):
    cid = lax.axis_index("c")
    sid = lax.axis_index("s")
    wid = sid * NC + cid

    c0 = pl.multiple_of(wid * TRIPS, 8)
    pltpu.sync_copy(dst.at[pl.ds(c0, TRIPS)], didx)

    nst = dout // 16

    def _zero_rows(i, _):
        r = i // nst
        col = (i % nst) * 16
        rows_v0[r, pl.ds(col, 16)] = jnp.zeros((16,), jnp.float32)
        return 0

    lax.fori_loop(0, CHUNK * nst, _zero_rows, 0)

    base_row = pl.multiple_of(sid * RPT, 128)
    for k in range(RPT // CHUNK):
        pltpu.sync_copy(rows_v0,
                        acc_sh.at[pl.ds(base_row + k * CHUNK, CHUNK)])

    plsc.subcore_barrier()

    def _g(sv, rv, j):
        pltpu.sync_copy(src.at[c0 + j], sv)
        pltpu.async_copy(g.at[sv], rv, sem)

    def _wait_g(sv, rv):
        pltpu.make_async_copy(g.at[sv], rv, sem).wait()

    def _s(rv, j):
        pltpu.async_copy(rv, acc_sh.at[didx.at[j]], sem2, add=True)

    def _wait_s(rv):
        pltpu.make_async_copy(rv, acc_sh.at[didx.at[0]], sem2).wait()

    _g(sv0, rows_v0, 0)
    _wait_g(sv0, rows_v0)
    _s(rows_v0, 0)
    _g(sv1, rows_v1, 1)

    def _pair(jj, _):
        j = 2 * jj + 1
        _wait_g(sv1, rows_v1)
        _s(rows_v1, j)
        _wait_s(rows_v0)
        _g(sv0, rows_v0, j + 1)
        _wait_g(sv0, rows_v0)
        _s(rows_v0, j + 1)
        _wait_s(rows_v1)
        _g(sv1, rows_v1, j + 2)
        return 0

    lax.fori_loop(0, TRIPS // 2 - 1, _pair, 0)

    _wait_g(sv1, rows_v1)
    _s(rows_v1, TRIPS - 1)
    _wait_s(rows_v0)
    _wait_s(rows_v1)

    plsc.subcore_barrier()

    for k in range(RPT // CHUNK):
        r0 = base_row + k * CHUNK
        pltpu.sync_copy(acc_sh.at[pl.ds(r0, CHUNK)],
                        acc_out.at[cid, pl.ds(r0, CHUNK)])


def _make_sc_agg(dout):
    mesh = plsc.VectorSubcoreMesh(core_axis_name="c", subcore_axis_name="s")
    scratch = [
        pltpu.VMEM((TRIPS, CHUNK), jnp.int32),
        pltpu.VMEM((CHUNK,), jnp.int32),
        pltpu.VMEM((CHUNK,), jnp.int32),
        pltpu.VMEM((CHUNK, dout), jnp.float32),
        pltpu.VMEM((CHUNK, dout), jnp.float32),
        pltpu.VMEM_SHARED((NP, dout), jnp.float32),
        pltpu.SemaphoreType.DMA,
        pltpu.SemaphoreType.DMA,
    ]

    def body(g, src, dst, acc_out, didx, sv0, sv1, rows_v0, rows_v1,
             acc_sh, sem, sem2):
        _agg_body(g, src, dst, acc_out, didx, sv0, sv1, rows_v0, rows_v1,
                  acc_sh, sem, sem2, dout)

    return pl.kernel(body, mesh=mesh,
                     out_type=jax.ShapeDtypeStruct((NC, NP, dout),
                                                   jnp.float32),
                     scratch_types=tuple(scratch))


def _cnt_body(dst, cnt_out, didx, ones_v, cnt_sh, sem):
    cid = lax.axis_index("c")
    sid = lax.axis_index("s")
    wid = sid * NC + cid

    c0 = pl.multiple_of(wid * TRIPS, 8)
    pltpu.sync_copy(dst.at[pl.ds(c0, TRIPS)], didx)

    def _zero(i, _):
        r = i // 8
        col = (i % 8) * 16
        ones_v[r, pl.ds(col, 16)] = jnp.zeros((16,), jnp.float32)
        return 0

    lax.fori_loop(0, CHUNK * 8, _zero, 0)

    base_row = pl.multiple_of(sid * RPT, 128)
    for k in range(RPT // CHUNK):
        pltpu.sync_copy(ones_v, cnt_sh.at[pl.ds(base_row + k * CHUNK, CHUNK)])

    def _fill(i, _):
        r = i // 8
        col = (i % 8) * 16
        ones_v[r, pl.ds(col, 16)] = jnp.full((16,), 1.0, jnp.float32)
        return 0

    lax.fori_loop(0, CHUNK * 8, _fill, 0)

    plsc.subcore_barrier()

    def _step(j, _):
        pltpu.sync_copy(ones_v, cnt_sh.at[didx.at[j]], add=True)
        return 0

    lax.fori_loop(0, TRIPS, _step, 0)

    plsc.subcore_barrier()

    for k in range(RPT // CHUNK):
        r0 = base_row + k * CHUNK
        pltpu.sync_copy(cnt_sh.at[pl.ds(r0, CHUNK)],
                        cnt_out.at[cid, pl.ds(r0, CHUNK)])


def _make_sc_cnt():
    mesh = plsc.VectorSubcoreMesh(core_axis_name="c", subcore_axis_name="s")
    scratch = [
        pltpu.VMEM((TRIPS, CHUNK), jnp.int32),
        pltpu.VMEM((CHUNK, H), jnp.float32),
        pltpu.VMEM_SHARED((NP, H), jnp.float32),
        pltpu.SemaphoreType.DMA,
    ]
    return pl.kernel(_cnt_body, mesh=mesh,
                     out_type=jax.ShapeDtypeStruct((NC, NP, H), jnp.float32),
                     scratch_types=tuple(scratch))


_sc_cache = {}


def _sc_agg(dout):
    if dout not in _sc_cache:
        _sc_cache[dout] = _make_sc_agg(dout)
    return _sc_cache[dout]


def _sc_cnt():
    if 'cnt' not in _sc_cache:
        _sc_cache['cnt'] = _make_sc_cnt()
    return _sc_cache['cnt']


def _pre_body(x_ref, wlt_ref, wrt_ref, g_ref, r_ref):
    xb = x_ref[...]
    g_ref[...] = jnp.dot(xb, wlt_ref[...], preferred_element_type=jnp.float32)
    r_ref[...] = jnp.dot(xb, wrt_ref[...], preferred_element_type=jnp.float32)


_pre = pl.pallas_call(
    _pre_body,
    grid=(GRID,),
    in_specs=[
        pl.BlockSpec((BM, D), lambda i: (i, 0)),
        pl.BlockSpec((D, H), lambda i: (0, 0)),
        pl.BlockSpec((D, H), lambda i: (0, 0)),
    ],
    out_specs=[pl.BlockSpec((BM, H), lambda i: (i, 0))] * 2,
    out_shape=[jax.ShapeDtypeStruct((NP, H), jnp.float32)] * 2,
)


def _comb1_body(p0, p1, c0, c1, bl, r, wlt, wrt, g2_ref, r2_ref, invc_ref):
    cnt = c0[...][:, 0:1] + c1[...][:, 0:1]
    invc = 1.0 / jnp.maximum(cnt, 1.0)
    h = jnp.maximum((p0[...] + p1[...]) * invc + bl[...] + r[...], 0.0)
    g2_ref[...] = jnp.dot(h, wlt[...], preferred_element_type=jnp.float32)
    r2_ref[...] = jnp.dot(h, wrt[...], preferred_element_type=jnp.float32)
    invc_ref[...] = jnp.broadcast_to(invc, invc_ref.shape)


_comb1 = pl.pallas_call(
    _comb1_body,
    grid=(GRID,),
    in_specs=[
        pl.BlockSpec((BM, H), lambda i: (i, 0)),
        pl.BlockSpec((BM, H), lambda i: (i, 0)),
        pl.BlockSpec((BM, H), lambda i: (i, 0)),
        pl.BlockSpec((BM, H), lambda i: (i, 0)),
        pl.BlockSpec((1, H), lambda i: (0, 0)),
        pl.BlockSpec((BM, H), lambda i: (i, 0)),
        pl.BlockSpec((H, H), lambda i: (0, 0)),
        pl.BlockSpec((H, H), lambda i: (0, 0)),
    ],
    out_specs=[
        pl.BlockSpec((BM, H), lambda i: (i, 0)),
        pl.BlockSpec((BM, H), lambda i: (i, 0)),
        pl.BlockSpec((BM, H), lambda i: (i, 0)),
    ],
    out_shape=[
        jax.ShapeDtypeStruct((NP, H), jnp.float32),
        jax.ShapeDtypeStruct((NP, H), jnp.float32),
        jax.ShapeDtypeStruct((NP, H), jnp.float32),
    ],
)


def _comb2_body(p0, p1, invc, bl, r, h3_ref):
    h3_ref[...] = jnp.maximum(
        (p0[...] + p1[...]) * invc[...] + bl[...] + r[...], 0.0)


_comb2 = pl.pallas_call(
    _comb2_body,
    grid=(GRID,),
    in_specs=[
        pl.BlockSpec((BM, H), lambda i: (i, 0)),
        pl.BlockSpec((BM, H), lambda i: (i, 0)),
        pl.BlockSpec((BM, H), lambda i: (i, 0)),
        pl.BlockSpec((1, H), lambda i: (0, 0)),
        pl.BlockSpec((BM, H), lambda i: (i, 0)),
    ],
    out_specs=pl.BlockSpec((BM, H), lambda i: (i, 0)),
    out_shape=jax.ShapeDtypeStruct((NP, H), jnp.float32),
)


def _comb3_body(p0, p1, invc, bl, h3, wlt, wrt, out_ref):
    agg = (p0[...] + p1[...]) * invc[...]
    out_ref[...] = jnp.maximum(
        jnp.dot(agg, wlt[...], preferred_element_type=jnp.float32)
        + bl[...]
        + jnp.dot(h3[...], wrt[...], preferred_element_type=jnp.float32),
        0.0)


_comb3 = pl.pallas_call(
    _comb3_body,
    grid=(GRID,),
    in_specs=[
        pl.BlockSpec((BM, H), lambda i: (i, 0)),
        pl.BlockSpec((BM, H), lambda i: (i, 0)),
        pl.BlockSpec((BM, H), lambda i: (i, 0)),
        pl.BlockSpec((1, C), lambda i: (0, 0)),
        pl.BlockSpec((BM, H), lambda i: (i, 0)),
        pl.BlockSpec((H, C), lambda i: (0, 0)),
        pl.BlockSpec((H, C), lambda i: (0, 0)),
    ],
    out_specs=pl.BlockSpec((BM, C), lambda i: (i, 0)),
    out_shape=jax.ShapeDtypeStruct((NP, C), jnp.float32),
)


def kernel(x, edge_index, Wl1, bl1, Wr1, Wl2, bl2, Wr2, Wl3, bl3, Wr3):
    pad = jnp.arange(EP - E, dtype=jnp.int32)
    src = jnp.concatenate([edge_index[0], pad % N]).reshape(NCHUNKS, CHUNK)
    dst = jnp.concatenate(
        [edge_index[1], N + pad % (NP - N)]).reshape(NCHUNKS, CHUNK)
    xp = jnp.pad(x, ((0, NP - N), (0, 0)))

    g1, r1 = _pre(xp, Wl1.T, Wr1.T)
    acc1 = _sc_agg(H)(g1, src, dst)
    cnts = _sc_cnt()(dst)
    g2, r2, invc = _comb1(acc1[0], acc1[1], cnts[0], cnts[1],
                          bl1.reshape(1, H), r1, Wl2.T, Wr2.T)
    acc2 = _sc_agg(H)(g2, src, dst)
    h3 = _comb2(acc2[0], acc2[1], invc, bl2.reshape(1, H), r2)
    acc3 = _sc_agg(H)(h3, src, dst)
    out = _comb3(acc3[0], acc3[1], invc, bl3.reshape(1, C), h3,
                 Wl3.T, Wr3.T)
    return out[:N]
